# baseline (speedup 1.0000x reference)
import functools

import jax
import jax.numpy as jnp
from jax.experimental import pallas as pl
from jax.experimental.pallas import tpu as pltpu

_NBUF = 3
_LOOKAHEAD = 2
_NSPLIT = 2


def _copies(adj_hbm, abuf, sem, blk, bm):
    slot = jax.lax.rem(blk, _NBUF)
    half = bm // _NSPLIT
    return [
        pltpu.make_async_copy(
            adj_hbm.at[pl.ds(blk * bm + i * half, half), :],
            abuf.at[slot, pl.ds(i * half, half)],
            sem.at[slot, i],
        )
        for i in range(_NSPLIT)
    ]


def _gcn_block(
    x_ref, ws_ref, wn_ref, b_ref, adj_hbm, out_ref, support_ref, abuf, sem, *, bm, nsteps
):
    m = pl.program_id(0)

    @pl.when(m == 0)
    def _():
        support_ref[...] = jnp.dot(
            x_ref[...], wn_ref[...], preferred_element_type=jnp.float32
        )
        for blk in range(_LOOKAHEAD):
            for c in _copies(adj_hbm, abuf, sem, blk, bm):
                c.start()

    nxt = m + _LOOKAHEAD

    @pl.when(nxt < nsteps)
    def _():
        for c in _copies(adj_hbm, abuf, sem, nxt, bm):
            c.start()

    for c in _copies(adj_hbm, abuf, sem, m, bm):
        c.wait()

    x_blk = x_ref[pl.ds(m * bm, bm), :]
    acc = jnp.dot(x_blk, ws_ref[...], preferred_element_type=jnp.float32)
    acc += jnp.dot(
        abuf[jax.lax.rem(m, _NBUF)], support_ref[...], preferred_element_type=jnp.float32
    )
    out_ref[...] = acc + b_ref[...]


def kernel(input, adj, weight_self, weight_neighbor, bias):
    n, d_in = input.shape
    d_out = weight_self.shape[1]
    bm = 400
    nsteps = n // bm
    return pl.pallas_call(
        functools.partial(_gcn_block, bm=bm, nsteps=nsteps),
        grid=(nsteps,),
        in_specs=[
            pl.BlockSpec((n, d_in), lambda m: (0, 0)),
            pl.BlockSpec((d_in, d_out), lambda m: (0, 0)),
            pl.BlockSpec((d_in, d_out), lambda m: (0, 0)),
            pl.BlockSpec((1, d_out), lambda m: (0, 0)),
            pl.BlockSpec(memory_space=pl.ANY),
        ],
        out_specs=pl.BlockSpec((bm, d_out), lambda m: (m, 0)),
        out_shape=jax.ShapeDtypeStruct((n, d_out), jnp.float32),
        scratch_shapes=[
            pltpu.VMEM((n, d_out), jnp.float32),
            pltpu.VMEM((_NBUF, bm, n), jnp.float32),
            pltpu.SemaphoreType.DMA((_NBUF, _NSPLIT)),
        ],
    )(input, weight_self, weight_neighbor, bias.reshape(1, -1), adj)


# speedup vs baseline: 1.0343x; 1.0343x over previous
import functools

import jax
import jax.numpy as jnp
from jax.experimental import pallas as pl
from jax.experimental.pallas import tpu as pltpu


def _gcn_block(x_ref, ws_ref, wn_ref, b_ref, adj_ref, out_ref, support_ref, *, bm):
    m = pl.program_id(0)

    @pl.when(m == 0)
    def _():
        support_ref[...] = jnp.dot(
            x_ref[...], wn_ref[...], preferred_element_type=jnp.float32
        )

    x_blk = x_ref[pl.ds(m * bm, bm), :]
    acc = jnp.dot(x_blk, ws_ref[...], preferred_element_type=jnp.float32)
    acc += jnp.dot(adj_ref[...], support_ref[...], preferred_element_type=jnp.float32)
    out_ref[...] = acc + b_ref[...]


def kernel(input, adj, weight_self, weight_neighbor, bias):
    n, d_in = input.shape
    d_out = weight_self.shape[1]
    bm = 400
    grid = (n // bm,)
    return pl.pallas_call(
        functools.partial(_gcn_block, bm=bm),
        grid=grid,
        in_specs=[
            pl.BlockSpec((n, d_in), lambda m: (0, 0)),
            pl.BlockSpec((d_in, d_out), lambda m: (0, 0)),
            pl.BlockSpec((d_in, d_out), lambda m: (0, 0)),
            pl.BlockSpec((1, d_out), lambda m: (0, 0)),
            pl.BlockSpec((bm, n), lambda m: (m, 0)),
        ],
        out_specs=pl.BlockSpec((bm, d_out), lambda m: (m, 0)),
        out_shape=jax.ShapeDtypeStruct((n, d_out), jnp.float32),
        scratch_shapes=[pltpu.VMEM((n, d_out), jnp.float32)],
    )(input, weight_self, weight_neighbor, bias.reshape(1, -1), adj)
